# baseline (speedup 1.0000x reference)
import functools

import jax
import jax.numpy as jnp
from jax.experimental import pallas as pl

_NUM_CENTROIDS = 512
_RADIUS_LIST = [0.1, 0.2, 0.4]
_NUM_SAMPLES_LIST = [16, 32, 64]
_CPAD = 128
_BLK = 512


def _sq_dist(src, dst):
    d = -2.0 * jnp.matmul(src, jnp.transpose(dst, (0, 2, 1)))
    d = d + jnp.sum(src ** 2, -1)[..., :, None]
    d = d + jnp.sum(dst ** 2, -1)[..., None, :]
    return d


def _index_points(points, idx):
    return jax.vmap(lambda p, i: p[i])(points, idx)


def _fps(xyz, num_centroids):
    B, N, C = xyz.shape
    batch = jnp.arange(B)
    farthest = jax.random.randint(jax.random.key(42), (B,), 0, N)
    distance = jnp.full((B, N), jnp.inf, dtype=xyz.dtype)
    centroid_inds = jnp.zeros((B, num_centroids), dtype=jnp.int32)

    def body(i, state):
        c_inds, dist_state, far = state
        c_inds = c_inds.at[:, i].set(far.astype(jnp.int32))
        centroid = xyz[batch, far, :].reshape(B, 1, C)
        d = jnp.sum((xyz - centroid) ** 2, -1)
        dist_state = jnp.minimum(dist_state, d)
        far = jnp.argmax(dist_state, axis=-1)
        return (c_inds, dist_state, far)

    centroid_inds, _, _ = jax.lax.fori_loop(
        0, num_centroids, body, (centroid_inds, distance, farthest))
    return centroid_inds


def _query_ball(radius, num_samples, xyz, query_xyz):
    B, N, _ = xyz.shape
    S = query_xyz.shape[1]
    group_inds = jnp.broadcast_to(jnp.arange(N, dtype=jnp.int32), (B, S, N))
    sqr_dists = _sq_dist(query_xyz, xyz)
    group_inds = jnp.where(sqr_dists > radius ** 2, N, group_inds)
    group_inds = jnp.sort(group_inds, axis=-1)[:, :, :num_samples]
    group_first = group_inds[:, :, :1]
    group_inds = jnp.where(group_inds == N, group_first, group_inds)
    return group_inds


def _mm_kernel(x_ref, w_ref, b_ref, y_ref, s_ref):
    i = pl.program_id(0)
    y = jnp.dot(x_ref[...], w_ref[...],
                preferred_element_type=jnp.float32) + b_ref[...]
    y_ref[...] = y

    @pl.when(i == 0)
    def _():
        s_ref[...] = jnp.zeros_like(s_ref)

    s_ref[0:1, :] += jnp.sum(y, axis=0, keepdims=True)
    s_ref[1:2, :] += jnp.sum(y * y, axis=0, keepdims=True)


def _bn_kernel(y_ref, s_ref, g_ref, bt_ref, o_ref, *, count, kmax):
    y = y_ref[...]
    mean = s_ref[0:1, :] / count
    var = s_ref[1:2, :] / count - mean * mean
    x = (y - mean) * jax.lax.rsqrt(var + 1e-5) * g_ref[...] + bt_ref[...]
    x = jnp.maximum(x, 0.0)
    if kmax is None:
        o_ref[...] = x
    else:
        o_ref[...] = jnp.max(x.reshape(-1, kmax, _CPAD), axis=1)


def _mlp_layer(x, wt, b, gamma, beta, *, last_k=None):
    P = x.shape[0]
    grid = P // _BLK
    y, s = pl.pallas_call(
        _mm_kernel,
        grid=(grid,),
        in_specs=[
            pl.BlockSpec((_BLK, _CPAD), lambda i: (i, 0)),
            pl.BlockSpec((_CPAD, _CPAD), lambda i: (0, 0)),
            pl.BlockSpec((1, _CPAD), lambda i: (0, 0)),
        ],
        out_specs=[
            pl.BlockSpec((_BLK, _CPAD), lambda i: (i, 0)),
            pl.BlockSpec((2, _CPAD), lambda i: (0, 0)),
        ],
        out_shape=[
            jax.ShapeDtypeStruct((P, _CPAD), jnp.float32),
            jax.ShapeDtypeStruct((2, _CPAD), jnp.float32),
        ],
    )(x, wt, b)
    if last_k is None:
        out_shape = jax.ShapeDtypeStruct((P, _CPAD), jnp.float32)
        out_spec = pl.BlockSpec((_BLK, _CPAD), lambda i: (i, 0))
    else:
        out_shape = jax.ShapeDtypeStruct((P // last_k, _CPAD), jnp.float32)
        out_spec = pl.BlockSpec((_BLK // last_k, _CPAD), lambda i: (i, 0))
    return pl.pallas_call(
        functools.partial(_bn_kernel, count=float(P), kmax=last_k),
        grid=(grid,),
        in_specs=[
            pl.BlockSpec((_BLK, _CPAD), lambda i: (i, 0)),
            pl.BlockSpec((2, _CPAD), lambda i: (0, 0)),
            pl.BlockSpec((1, _CPAD), lambda i: (0, 0)),
            pl.BlockSpec((1, _CPAD), lambda i: (0, 0)),
        ],
        out_specs=out_spec,
        out_shape=out_shape,
    )(y, s, gamma, beta)


def _pad_params(params):
    padded = []
    for p in params:
        cout, cin = p["W"].shape
        wt = jnp.zeros((_CPAD, _CPAD), jnp.float32).at[:cin, :cout].set(p["W"].T)
        b = jnp.zeros((1, _CPAD), jnp.float32).at[0, :cout].set(p["b"])
        g = jnp.zeros((1, _CPAD), jnp.float32).at[0, :cout].set(p["gamma"])
        bt = jnp.zeros((1, _CPAD), jnp.float32).at[0, :cout].set(p["beta"])
        padded.append((wt, b, g, bt))
    return padded


def kernel(points_xyz, points_features, params):
    xyz = jnp.transpose(points_xyz, (0, 2, 1))
    feats = jnp.transpose(points_features, (0, 2, 1))
    B, N, _ = xyz.shape
    S = _NUM_CENTROIDS
    xyz_sg = jax.lax.stop_gradient(xyz)
    centroid_inds = _fps(xyz_sg, S)
    centroids_xyz = _index_points(xyz, centroid_inds)
    padded_params = _pad_params(params)
    cin0 = feats.shape[-1] + 3
    outs = []
    for radius, K in zip(_RADIUS_LIST, _NUM_SAMPLES_LIST):
        group_inds = _query_ball(radius, K, xyz_sg,
                                 jax.lax.stop_gradient(centroids_xyz))
        grouped_xyz = _index_points(xyz, group_inds) - centroids_xyz[:, :, None, :]
        grouped_feats = _index_points(feats, group_inds)
        grouped = jnp.concatenate([grouped_feats, grouped_xyz], axis=-1)
        P = B * S * K
        x = jnp.zeros((P, _CPAD), jnp.float32)
        x = x.at[:, :cin0].set(grouped.reshape(P, cin0))
        for li, (wt, b, g, bt) in enumerate(padded_params):
            last = K if li == len(padded_params) - 1 else None
            x = _mlp_layer(x, wt, b, g, bt, last_k=last)
        outs.append(jnp.transpose(x.reshape(B, S, _CPAD), (0, 2, 1)))
    return (jnp.transpose(centroids_xyz, (0, 2, 1)),
            jnp.concatenate(outs, axis=1))


# speedup vs baseline: 1.0785x; 1.0785x over previous
import functools

import jax
import jax.numpy as jnp
from jax.experimental import pallas as pl

_NUM_CENTROIDS = 512
_RADIUS_LIST = [0.1, 0.2, 0.4]
_NUM_SAMPLES_LIST = [16, 32, 64]
_CPAD = 128
_BLK = 512


def _sq_dist(src, dst):
    d = -2.0 * jnp.matmul(src, jnp.transpose(dst, (0, 2, 1)))
    d = d + jnp.sum(src ** 2, -1)[..., :, None]
    d = d + jnp.sum(dst ** 2, -1)[..., None, :]
    return d


def _index_points(points, idx):
    return jax.vmap(lambda p, i: p[i])(points, idx)


def _fps(xyz, num_centroids):
    B, N, C = xyz.shape
    batch = jnp.arange(B)
    farthest = jax.random.randint(jax.random.key(42), (B,), 0, N)
    distance = jnp.full((B, N), jnp.inf, dtype=xyz.dtype)
    centroid_inds = jnp.zeros((B, num_centroids), dtype=jnp.int32)

    def body(i, state):
        c_inds, dist_state, far = state
        c_inds = c_inds.at[:, i].set(far.astype(jnp.int32))
        centroid = xyz[batch, far, :].reshape(B, 1, C)
        d = jnp.sum((xyz - centroid) ** 2, -1)
        dist_state = jnp.minimum(dist_state, d)
        far = jnp.argmax(dist_state, axis=-1)
        return (c_inds, dist_state, far)

    centroid_inds, _, _ = jax.lax.fori_loop(
        0, num_centroids, body, (centroid_inds, distance, farthest))
    return centroid_inds


def _query_ball(radius, num_samples, xyz, query_xyz):
    B, N, _ = xyz.shape
    S = query_xyz.shape[1]
    group_inds = jnp.broadcast_to(jnp.arange(N, dtype=jnp.int32), (B, S, N))
    sqr_dists = _sq_dist(query_xyz, xyz)
    group_inds = jnp.where(sqr_dists > radius ** 2, N, group_inds)
    group_inds = jnp.sort(group_inds, axis=-1)[:, :, :num_samples]
    group_first = group_inds[:, :, :1]
    group_inds = jnp.where(group_inds == N, group_first, group_inds)
    return group_inds


def _mm_kernel(x_ref, w_ref, b_ref, y_ref, s_ref):
    i = pl.program_id(0)
    y = jnp.dot(x_ref[...], w_ref[...],
                preferred_element_type=jnp.float32) + b_ref[...]
    y_ref[...] = y

    @pl.when(i == 0)
    def _():
        s_ref[...] = jnp.zeros_like(s_ref)

    s_ref[0:1, :] += jnp.sum(y, axis=0, keepdims=True)
    s_ref[1:2, :] += jnp.sum(y * y, axis=0, keepdims=True)


def _mm_fused_kernel(x_ref, sp_ref, g_ref, bt_ref, w_ref, b_ref, y_ref,
                     s_ref, *, count):
    i = pl.program_id(0)
    mean = sp_ref[0:1, :] / count
    var = sp_ref[1:2, :] / count - mean * mean
    x = (x_ref[...] - mean) * jax.lax.rsqrt(var + 1e-5) * g_ref[...] + bt_ref[...]
    x = jnp.maximum(x, 0.0)
    y = jnp.dot(x, w_ref[...], preferred_element_type=jnp.float32) + b_ref[...]
    y_ref[...] = y

    @pl.when(i == 0)
    def _():
        s_ref[...] = jnp.zeros_like(s_ref)

    s_ref[0:1, :] += jnp.sum(y, axis=0, keepdims=True)
    s_ref[1:2, :] += jnp.sum(y * y, axis=0, keepdims=True)


def _bn_kernel(y_ref, s_ref, g_ref, bt_ref, o_ref, *, count, kmax):
    y = y_ref[...]
    mean = s_ref[0:1, :] / count
    var = s_ref[1:2, :] / count - mean * mean
    x = (y - mean) * jax.lax.rsqrt(var + 1e-5) * g_ref[...] + bt_ref[...]
    x = jnp.maximum(x, 0.0)
    if kmax is None:
        o_ref[...] = x
    else:
        o_ref[...] = jnp.max(x.reshape(-1, kmax, _CPAD), axis=1)


def _row_spec(rows):
    return pl.BlockSpec((rows, _CPAD), lambda i: (i, 0))


def _bcast_spec(rows):
    return pl.BlockSpec((rows, _CPAD), lambda i: (0, 0))


def _mlp_chain(x, padded_params, K):
    P = x.shape[0]
    grid = P // _BLK
    mm_outs = [
        jax.ShapeDtypeStruct((P, _CPAD), jnp.float32),
        jax.ShapeDtypeStruct((2, _CPAD), jnp.float32),
    ]
    mm_out_specs = [_row_spec(_BLK), _bcast_spec(2)]
    wt, b, _, _ = padded_params[0]
    y, s = pl.pallas_call(
        _mm_kernel,
        grid=(grid,),
        in_specs=[_row_spec(_BLK), _bcast_spec(_CPAD), _bcast_spec(1)],
        out_specs=mm_out_specs,
        out_shape=mm_outs,
    )(x, wt, b)
    for li in range(1, len(padded_params)):
        wt, b, _, _ = padded_params[li]
        _, _, g_prev, bt_prev = padded_params[li - 1]
        y, s = pl.pallas_call(
            functools.partial(_mm_fused_kernel, count=float(P)),
            grid=(grid,),
            in_specs=[_row_spec(_BLK), _bcast_spec(2), _bcast_spec(1),
                      _bcast_spec(1), _bcast_spec(_CPAD), _bcast_spec(1)],
            out_specs=mm_out_specs,
            out_shape=mm_outs,
        )(y, s, g_prev, bt_prev, wt, b)
    _, _, g, bt = padded_params[-1]
    return pl.pallas_call(
        functools.partial(_bn_kernel, count=float(P), kmax=K),
        grid=(grid,),
        in_specs=[_row_spec(_BLK), _bcast_spec(2), _bcast_spec(1),
                  _bcast_spec(1)],
        out_specs=_row_spec(_BLK // K),
        out_shape=jax.ShapeDtypeStruct((P // K, _CPAD), jnp.float32),
    )(y, s, g, bt)


def _pad_params(params):
    padded = []
    for p in params:
        cout, cin = p["W"].shape
        wt = jnp.zeros((_CPAD, _CPAD), jnp.float32).at[:cin, :cout].set(p["W"].T)
        b = jnp.zeros((1, _CPAD), jnp.float32).at[0, :cout].set(p["b"])
        g = jnp.zeros((1, _CPAD), jnp.float32).at[0, :cout].set(p["gamma"])
        bt = jnp.zeros((1, _CPAD), jnp.float32).at[0, :cout].set(p["beta"])
        padded.append((wt, b, g, bt))
    return padded


def kernel(points_xyz, points_features, params):
    xyz = jnp.transpose(points_xyz, (0, 2, 1))
    feats = jnp.transpose(points_features, (0, 2, 1))
    B, N, _ = xyz.shape
    S = _NUM_CENTROIDS
    xyz_sg = jax.lax.stop_gradient(xyz)
    centroid_inds = _fps(xyz_sg, S)
    centroids_xyz = _index_points(xyz, centroid_inds)
    padded_params = _pad_params(params)
    cin0 = feats.shape[-1] + 3
    outs = []
    for radius, K in zip(_RADIUS_LIST, _NUM_SAMPLES_LIST):
        group_inds = _query_ball(radius, K, xyz_sg,
                                 jax.lax.stop_gradient(centroids_xyz))
        grouped_xyz = _index_points(xyz, group_inds) - centroids_xyz[:, :, None, :]
        grouped_feats = _index_points(feats, group_inds)
        grouped = jnp.concatenate([grouped_feats, grouped_xyz], axis=-1)
        P = B * S * K
        x = jnp.zeros((P, _CPAD), jnp.float32)
        x = x.at[:, :cin0].set(grouped.reshape(P, cin0))
        x = _mlp_chain(x, padded_params, K)
        outs.append(jnp.transpose(x.reshape(B, S, _CPAD), (0, 2, 1)))
    return (jnp.transpose(centroids_xyz, (0, 2, 1)),
            jnp.concatenate(outs, axis=1))
